# baseline (speedup 1.0000x reference)
import jax
import jax.numpy as jnp
from jax import lax
from jax.experimental import pallas as pl
from jax.experimental.pallas import tpu as pltpu
from jax.experimental.pallas import tpu_sc as plsc

_NUM_SEQ = 100000
_NUM_FRAMES = 20
_DIM = 32
_BATCH = 4096
_SEL = 20

_NC = 2
_NS = 16
_NW = _NC * _NS
_B_PER_W = _BATCH // _NW
_P_PER_W = _B_PER_W * _SEL
_GCHUNK = 128
_NCHUNK = _P_PER_W // _GCHUNK

_DIV_MAGIC = 3277
_DIV_SHIFT = 16


def _body(table_hbm, seq_hbm, frame_hbm, out_hbm, seq_v, frm_v, idx_v, rows_v, sem):
    wid = lax.axis_index("s") * _NC + lax.axis_index("c")
    pbase = wid * _P_PER_W
    bbase = wid * _B_PER_W

    pltpu.sync_copy(seq_hbm.at[pl.ds(bbase, _B_PER_W)], seq_v)
    pltpu.sync_copy(frame_hbm.at[pl.ds(pbase, _P_PER_W)], frm_v)

    lane = lax.iota(jnp.int32, 16)
    c_magic = jnp.full((16,), _DIV_MAGIC, jnp.int32)
    c_shift = jnp.full((16,), _DIV_SHIFT, jnp.int32)
    c_nf = jnp.full((16,), _NUM_FRAMES, jnp.int32)
    c_step = jnp.full((16,), 16, jnp.int32)

    @pl.loop(0, _P_PER_W // 16, init_carry=lane)
    def _compute_idx(i, p):
        b = lax.shift_right_logical(p * c_magic, c_shift)
        seq = plsc.load_gather(seq_v, [b])
        f = frm_v[pl.ds(i * 16, 16)]
        j = i // (_GCHUNK // 16)
        col = (i % (_GCHUNK // 16)) * 16
        idx_v[j, pl.ds(col, 16)] = seq * c_nf + f
        return p + c_step

    copies = []
    for j in range(_NCHUNK):
        copies.append(
            pltpu.async_copy(
                table_hbm.at[idx_v.at[j]],
                rows_v.at[pl.ds(j * _GCHUNK, _GCHUNK)],
                sem,
            )
        )
    for c in copies:
        c.wait()

    c_zero = jnp.zeros((16,), jnp.float32)

    @pl.loop(0, _P_PER_W * (_DIM // 16))
    def _relu(i):
        r = i // (_DIM // 16)
        c = (i % (_DIM // 16)) * 16
        rows_v[r, pl.ds(c, 16)] = jnp.maximum(rows_v[r, pl.ds(c, 16)], c_zero)

    pltpu.sync_copy(rows_v, out_hbm.at[pl.ds(pbase, _P_PER_W)])


@jax.jit
def kernel(table, seq_idx, frame_idx):
    table_flat = table.reshape(_NUM_SEQ * _NUM_FRAMES, _DIM)
    frames_flat = frame_idx.reshape(_BATCH * _SEL)
    mesh = plsc.VectorSubcoreMesh(core_axis_name="c", subcore_axis_name="s")
    out = pl.kernel(
        _body,
        out_type=jax.ShapeDtypeStruct((_BATCH * _SEL, _DIM), jnp.float32),
        mesh=mesh,
        compiler_params=pltpu.CompilerParams(
            use_tc_tiling_on_sc=False, needs_layout_passes=False),
        scratch_types=[
            pltpu.VMEM((_B_PER_W,), jnp.int32),
            pltpu.VMEM((_P_PER_W,), jnp.int32),
            pltpu.VMEM((_NCHUNK, _GCHUNK), jnp.int32),
            pltpu.VMEM((_P_PER_W, _DIM), jnp.float32),
            pltpu.SemaphoreType.DMA,
        ],
    )(table_flat, seq_idx, frames_flat)
    return out.reshape(_BATCH, _SEL, _DIM)
